# baseline (speedup 1.0000x reference)
import functools

import jax
import jax.numpy as jnp
from jax import lax
from jax.experimental import pallas as pl
from jax.experimental.pallas import tpu as pltpu
from jax.experimental.pallas import tpu_sc as plsc

_ALPHA = 0.5
_N_EDGES = 6400000
_NC, _NS, _L = 2, 16, 16
_NW = _NC * _NS
_PER_W = _N_EDGES // _NW
_CHUNK = 20000
_NCHUNK = _PER_W // _CHUNK
_NBUF = 2


@functools.partial(
    pl.kernel,
    out_type=jax.ShapeDtypeStruct((_N_EDGES,), jnp.float32),
    mesh=plsc.VectorSubcoreMesh(core_axis_name="c", subcore_axis_name="s"),
    scratch_types=[
        pltpu.VMEM((_CHUNK,), jnp.float32),
        pltpu.VMEM((_CHUNK,), jnp.float32),
        pltpu.VMEM((_CHUNK,), jnp.float32),
        pltpu.VMEM((_CHUNK,), jnp.float32),
        pltpu.SemaphoreType.DMA,
        pltpu.SemaphoreType.DMA,
        pltpu.SemaphoreType.DMA,
        pltpu.SemaphoreType.DMA,
    ],
)
def _elu_sc(w_hbm, out_hbm, w_v0, w_v1, o_v0, o_v1, si0, si1, so0, so1):
    wid = lax.axis_index("s") * _NC + lax.axis_index("c")
    base = wid * _PER_W
    w_bufs, o_bufs = (w_v0, w_v1), (o_v0, o_v1)
    in_sems, out_sems = (si0, si1), (so0, so1)

    in_d = [None] * _NCHUNK
    out_d = [None] * _NCHUNK
    for ci in range(_NBUF):
        off = base + ci * _CHUNK
        in_d[ci] = pltpu.async_copy(
            w_hbm.at[pl.ds(off, _CHUNK)], w_bufs[ci], in_sems[ci])

    for ci in range(_NCHUNK):
        b = ci % _NBUF
        off = base + ci * _CHUNK
        in_d[ci].wait()
        if ci >= _NBUF:
            out_d[ci - _NBUF].wait()
        w_v, o_v = w_bufs[b], o_bufs[b]

        @plsc.parallel_loop(0, _CHUNK, step=_L, unroll=16)
        def _vec(i):
            x = w_v[pl.ds(i, _L)] * _ALPHA - _ALPHA
            o_v[pl.ds(i, _L)] = jnp.where(x > 0.0, x + 1.0, jnp.exp(x))

        out_d[ci] = pltpu.async_copy(
            o_v, out_hbm.at[pl.ds(off, _CHUNK)], out_sems[b])
        nci = ci + _NBUF
        if nci < _NCHUNK:
            noff = base + nci * _CHUNK
            in_d[nci] = pltpu.async_copy(
                w_hbm.at[pl.ds(noff, _CHUNK)], w_bufs[b], in_sems[b])

    out_d[_NCHUNK - 2].wait()
    out_d[_NCHUNK - 1].wait()


_CB = 320000
_NBLK = _N_EDGES // _CB
_NBLK_A = 5


def _copy_body(x_ref, o_ref):
    o_ref[...] = x_ref[...]


def _copy_b_body(x_ref, a_ref, o_ref):
    del a_ref
    o_ref[...] = x_ref[...]


_copy_a = pl.pallas_call(
    _copy_body,
    grid=(_NBLK_A,),
    in_specs=[pl.BlockSpec((2, _CB), lambda i: (0, i))],
    out_specs=pl.BlockSpec((2, _CB), lambda i: (0, i)),
    out_shape=jax.ShapeDtypeStruct((2, _N_EDGES), jnp.int32),
)

_copy_b = pl.pallas_call(
    _copy_b_body,
    grid=(_NBLK - _NBLK_A,),
    in_specs=[
        pl.BlockSpec((2, _CB), lambda i: (0, i + _NBLK_A)),
        pl.BlockSpec(memory_space=pl.ANY),
    ],
    out_specs=pl.BlockSpec((2, _CB), lambda i: (0, i + _NBLK_A)),
    out_shape=jax.ShapeDtypeStruct((2, _N_EDGES), jnp.int32),
    input_output_aliases={1: 0},
)


def kernel(edge_index, edge_weight):
    idx_a = _copy_a(edge_index)
    edge_weight, idx_a = lax.optimization_barrier((edge_weight, idx_a))
    weights = _elu_sc(edge_weight)
    idx_full = _copy_b(edge_index, idx_a)
    return (idx_full, weights)


# speedup vs baseline: 1.0383x; 1.0383x over previous
import functools

import jax
import jax.numpy as jnp
from jax import lax
from jax.experimental import pallas as pl
from jax.experimental.pallas import tpu as pltpu
from jax.experimental.pallas import tpu_sc as plsc

_ALPHA = 0.5
_N_EDGES = 6400000
_NC, _NS, _L = 2, 16, 16
_NW = _NC * _NS
_PER_W = _N_EDGES // _NW
_CHUNK = 40000
_NCHUNK = _PER_W // _CHUNK
_NBUF = 3


@functools.partial(
    pl.kernel,
    out_type=jax.ShapeDtypeStruct((_N_EDGES,), jnp.float32),
    mesh=plsc.VectorSubcoreMesh(core_axis_name="c", subcore_axis_name="s"),
    scratch_types=[
        pltpu.VMEM((_CHUNK,), jnp.float32),
        pltpu.VMEM((_CHUNK,), jnp.float32),
        pltpu.VMEM((_CHUNK,), jnp.float32),
        pltpu.SemaphoreType.DMA,
        pltpu.SemaphoreType.DMA,
        pltpu.SemaphoreType.DMA,
        pltpu.SemaphoreType.DMA,
        pltpu.SemaphoreType.DMA,
        pltpu.SemaphoreType.DMA,
    ],
)
def _elu_sc(w_hbm, out_hbm, v0, v1, v2, si0, si1, si2, so0, so1, so2):
    wid = lax.axis_index("s") * _NC + lax.axis_index("c")
    base = wid * _PER_W
    bufs = (v0, v1, v2)
    in_sems, out_sems = (si0, si1, si2), (so0, so1, so2)

    in_d = [None] * _NCHUNK
    out_d = [None] * _NCHUNK
    for ci in range(_NBUF):
        off = base + ci * _CHUNK
        in_d[ci] = pltpu.async_copy(
            w_hbm.at[pl.ds(off, _CHUNK)], bufs[ci], in_sems[ci])

    for ci in range(_NCHUNK):
        b = ci % _NBUF
        off = base + ci * _CHUNK
        in_d[ci].wait()
        v = bufs[b]

        @plsc.parallel_loop(0, _CHUNK, step=_L, unroll=8)
        def _vec(i):
            x = v[pl.ds(i, _L)] * _ALPHA - _ALPHA
            v[pl.ds(i, _L)] = (jnp.exp(jnp.minimum(x, 0.0))
                               + jnp.maximum(x, 0.0))

        out_d[ci] = pltpu.async_copy(
            v, out_hbm.at[pl.ds(off, _CHUNK)], out_sems[b])
        pci, nci = ci - 1, ci - 1 + _NBUF
        if pci >= 0 and nci < _NCHUNK:
            out_d[pci].wait()
            noff = base + nci * _CHUNK
            in_d[nci] = pltpu.async_copy(
                w_hbm.at[pl.ds(noff, _CHUNK)], bufs[pci % _NBUF],
                in_sems[pci % _NBUF])

    for ci in range(_NCHUNK):
        if not (ci + 1 + _NBUF <= _NCHUNK):
            out_d[ci].wait()


_CB = 640000
_NBLK = _N_EDGES // _CB


def _copy_body(x_ref, o_ref):
    o_ref[...] = x_ref[...]


_tc_copy = pl.pallas_call(
    _copy_body,
    grid=(_NBLK,),
    in_specs=[pl.BlockSpec((2, _CB), lambda i: (0, i))],
    out_specs=pl.BlockSpec((2, _CB), lambda i: (0, i)),
    out_shape=jax.ShapeDtypeStruct((2, _N_EDGES), jnp.int32),
)


def kernel(edge_index, edge_weight):
    return (_tc_copy(edge_index), _elu_sc(edge_weight))


# speedup vs baseline: 1.0725x; 1.0329x over previous
import functools

import jax
import jax.numpy as jnp
from jax import lax
from jax.experimental import pallas as pl
from jax.experimental.pallas import tpu as pltpu
from jax.experimental.pallas import tpu_sc as plsc

_ALPHA = 0.5
_N_EDGES = 6400000
_NC, _NS, _L = 2, 16, 16
_NW = _NC * _NS
_PER_W = _N_EDGES // _NW
_CHUNK = 20000
_NCHUNK = _PER_W // _CHUNK
_NBUF = 2


@functools.partial(
    pl.kernel,
    out_type=jax.ShapeDtypeStruct((_N_EDGES,), jnp.float32),
    mesh=plsc.VectorSubcoreMesh(core_axis_name="c", subcore_axis_name="s"),
    scratch_types=[
        pltpu.VMEM((_CHUNK,), jnp.float32),
        pltpu.VMEM((_CHUNK,), jnp.float32),
        pltpu.VMEM((_CHUNK,), jnp.float32),
        pltpu.VMEM((_CHUNK,), jnp.float32),
        pltpu.SemaphoreType.DMA,
        pltpu.SemaphoreType.DMA,
        pltpu.SemaphoreType.DMA,
        pltpu.SemaphoreType.DMA,
    ],
)
def _elu_sc(w_hbm, out_hbm, w_v0, w_v1, o_v0, o_v1, si0, si1, so0, so1):
    wid = lax.axis_index("s") * _NC + lax.axis_index("c")
    base = wid * _PER_W
    w_bufs, o_bufs = (w_v0, w_v1), (o_v0, o_v1)
    in_sems, out_sems = (si0, si1), (so0, so1)

    in_d = [None] * _NCHUNK
    out_d = [None] * _NCHUNK
    for ci in range(_NBUF):
        off = base + ci * _CHUNK
        in_d[ci] = pltpu.async_copy(
            w_hbm.at[pl.ds(off, _CHUNK)], w_bufs[ci], in_sems[ci])

    for ci in range(_NCHUNK):
        b = ci % _NBUF
        off = base + ci * _CHUNK
        in_d[ci].wait()
        if ci >= _NBUF:
            out_d[ci - _NBUF].wait()
        w_v, o_v = w_bufs[b], o_bufs[b]

        @plsc.parallel_loop(0, _CHUNK, step=_L, unroll=8)
        def _vec(i):
            x = w_v[pl.ds(i, _L)] * _ALPHA - _ALPHA
            o_v[pl.ds(i, _L)] = (jnp.exp(jnp.minimum(x, 0.0))
                                 + jnp.maximum(x, 0.0))

        out_d[ci] = pltpu.async_copy(
            o_v, out_hbm.at[pl.ds(off, _CHUNK)], out_sems[b])
        nci = ci + _NBUF
        if nci < _NCHUNK:
            noff = base + nci * _CHUNK
            in_d[nci] = pltpu.async_copy(
                w_hbm.at[pl.ds(noff, _CHUNK)], w_bufs[b], in_sems[b])

    out_d[_NCHUNK - 2].wait()
    out_d[_NCHUNK - 1].wait()


_CB = 1280000
_NBLK = _N_EDGES // _CB


def _copy_body(x_ref, o_ref):
    o_ref[...] = x_ref[...]


_tc_copy = pl.pallas_call(
    _copy_body,
    grid=(_NBLK,),
    in_specs=[pl.BlockSpec((2, _CB), lambda i: (0, i))],
    out_specs=pl.BlockSpec((2, _CB), lambda i: (0, i)),
    out_shape=jax.ShapeDtypeStruct((2, _N_EDGES), jnp.int32),
)


def kernel(edge_index, edge_weight):
    return (_tc_copy(edge_index), _elu_sc(edge_weight))
